# Initial kernel scaffold; baseline (speedup 1.0000x reference)
import jax
import jax.numpy as jnp
from jax.experimental import pallas as pl


def kernel(x, fc_weight, bias):
    raise NotImplementedError("write your pallas kernel here")


# speedup vs baseline: 1.4897x; 1.4897x over previous
import jax
import jax.numpy as jnp
from jax import lax
from jax.experimental import pallas as pl
from jax.experimental.pallas import tpu as pltpu
from jax.experimental.pallas import tpu_sc as plsc

B = 16384
F = 26
FIELD_SIZE = 40000
NUM_WORKERS = 32
BPW = B // NUM_WORKERS
CHUNK = BPW * F
NVEC = BPW // 16


def _sc_body(xt_hbm, w_hbm, out_hbm, x_v, idx_v, g_v, o_v, sem):
  wid = lax.axis_index("s") * 2 + lax.axis_index("c")
  base = wid * BPW

  pltpu.sync_copy(xt_hbm.at[:, pl.ds(base, BPW)], x_v)

  def build_field(f, _):
    off = f * FIELD_SIZE
    for v in range(NVEC):
      vec = x_v[f, pl.ds(v * 16, 16)] + off
      idx_v[pl.ds(f * BPW + v * 16, 16)] = vec
    return 0

  lax.fori_loop(0, F, build_field, 0, unroll=False)

  pltpu.async_copy(w_hbm.at[idx_v], g_v, sem).wait()

  def reduce_vec(v, _):
    acc = g_v[pl.ds(v * 16, 16)]
    for f in range(1, F):
      acc = acc + g_v[pl.ds(f * BPW + v * 16, 16)]
    o_v[pl.ds(v * 16, 16)] = acc
    return 0

  lax.fori_loop(0, NVEC, reduce_vec, 0, unroll=False)

  pltpu.sync_copy(o_v, out_hbm.at[pl.ds(base, BPW)])


@jax.jit
def _sc_lookup(xt, w_flat):
  mesh = plsc.VectorSubcoreMesh(
      core_axis_name="c", subcore_axis_name="s", num_cores=2, num_subcores=16
  )
  return pl.kernel(
      _sc_body,
      out_type=jax.ShapeDtypeStruct((B,), jnp.float32),
      mesh=mesh,
      scratch_types=[
          pltpu.VMEM((F, BPW), jnp.int32),
          pltpu.VMEM((CHUNK,), jnp.int32),
          pltpu.VMEM((CHUNK,), jnp.float32),
          pltpu.VMEM((BPW,), jnp.float32),
          pltpu.SemaphoreType.DMA,
      ],
  )(xt, w_flat)


def kernel(x, fc_weight, bias):
  out = _sc_lookup(x.T, fc_weight.reshape(-1))
  return out.reshape(B, 1) + bias[None, :]
